# baseline (speedup 1.0000x reference)
import functools

import jax
import jax.numpy as jnp
from jax import lax
from jax.experimental import pallas as pl
from jax.experimental.pallas import tpu as pltpu
from jax.experimental.pallas import tpu_sc as plsc

N = 10000
NPAD = 10240
D = 128
E = 320000
NC = 2
NS = 16
NW = NC * NS
EPW = E // NW
EPT = E // NS
NHALF = NPAD // NC
KP = 120
MAXCH = 170
KEPT = MAXCH * KP
ACCR = NHALF + NS
TRASH = NHALF
RZ = ACCR // NS
RE = NHALF // NS
RPT = NPAD // NW

_f32 = jnp.float32
_i32 = jnp.int32


def _mesh():
    return plsc.VectorSubcoreMesh(core_axis_name="c", subcore_axis_name="s")


def _cparams():
    return pltpu.CompilerParams(needs_layout_passes=False,
                                use_tc_tiling_on_sc=False)


def _wid():
    return lax.axis_index("c") * NS + lax.axis_index("s")


def _rsqrt16(x):
    i = plsc.bitcast(x, _i32)
    i = jnp.int32(0x5F3759DF) - (i >> 1)
    y = plsc.bitcast(i, _f32)
    for _ in range(3):
        y = y * (1.5 - 0.5 * x * y * y)
    return y


def _splat16(vec_ref, idx):
    return plsc.load_gather(vec_ref, [jnp.zeros((16,), _i32) + idx])


def _edges_body(src_hbm, dst_hbm, hs_out, hd_out, ks_out, kd_out, cnt_out,
                src_in, dst_in, hist_v, ks_v, kd_v, cnt_v):
    c = lax.axis_index("c")
    s = lax.axis_index("s")
    w = c * NS + s
    iota = lax.iota(_i32, 16)
    ones = jnp.ones((16,), _f32)
    zeros = jnp.zeros((16,), _f32)
    full = iota >= 0

    pltpu.sync_copy(src_hbm.at[s], src_in)
    pltpu.sync_copy(dst_hbm.at[s], dst_in)

    hoff = c * EPW
    for buf, out_ref in ((src_in, hs_out), (dst_in, hd_out)):
        @pl.loop(0, NPAD // 16)
        def _(j):
            hist_v[pl.ds(j * 16, 16)] = zeros

        @pl.loop(0, EPW // 16)
        def _(g):
            idx16 = buf[pl.ds(hoff + g * 16, 16)]
            for lane in range(16):
                plsc.addupdate_scatter(hist_v, [idx16], ones,
                                       mask=iota == lane)

        pltpu.sync_copy(hist_v, out_ref.at[pl.ds(w * NPAD, NPAD)])

    lo = c * NHALF

    @pl.loop(0, EPT // 16, init_carry=jnp.int32(0))
    def pos(g, pos):
        sl = pl.ds(g * 16, 16)
        s16 = src_in[sl]
        d16 = dst_in[sl]
        mi = ((d16 >= lo) & (d16 < lo + NHALF)).astype(_i32)
        csum = plsc.cumsum(mi)
        dest = pos + csum - mi
        m = mi > 0
        plsc.store_scatter(ks_v, [dest], s16, mask=m)
        plsc.store_scatter(kd_v, [dest], d16 - lo, mask=m)
        return pos + csum[15]

    zi16 = jnp.zeros((16,), _i32)
    trash16 = zi16 + (TRASH + s)
    for g in range(15):
        dest = pos + g * 16 + iota
        plsc.store_scatter(ks_v, [dest], zi16, mask=full)
        plsc.store_scatter(kd_v, [dest], trash16, mask=full)

    nch = ((pos + (2 * KP - 1)) // (2 * KP)) * 2
    cnt_v[pl.ds(0, 16)] = zi16 + nch
    pltpu.sync_copy(cnt_v, cnt_out.at[pl.ds(w * 16, 16)])
    pltpu.sync_copy(ks_v, ks_out.at[pl.ds(w * KEPT, KEPT)])
    pltpu.sync_copy(kd_v, kd_out.at[pl.ds(w * KEPT, KEPT)])


def _edges_call(srcE, dstE):
    f = functools.partial(
        pl.kernel,
        out_type=[jax.ShapeDtypeStruct((NW * NPAD,), _f32),
                  jax.ShapeDtypeStruct((NW * NPAD,), _f32),
                  jax.ShapeDtypeStruct((NW * KEPT,), _i32),
                  jax.ShapeDtypeStruct((NW * KEPT,), _i32),
                  jax.ShapeDtypeStruct((NW * 16,), _i32)],
        mesh=_mesh(),
        compiler_params=_cparams(),
        scratch_types=[pltpu.VMEM((EPT,), _i32),
                       pltpu.VMEM((EPT,), _i32),
                       pltpu.VMEM((NPAD,), _f32),
                       pltpu.VMEM((KEPT,), _i32),
                       pltpu.VMEM((KEPT,), _i32),
                       pltpu.VMEM((16,), _i32)],
    )(_edges_body)
    return f(srcE, dstE)


def _prep_body(hs_hbm, hd_hbm, x_hbm, rsd_out, cc_out, g0_out,
               h2_v, rss_v, rsd_v, cc_v, xbuf, bsem):
    w = _wid()
    base = w * RPT

    for h_hbm, rs_v in ((hs_hbm, rss_v), (hd_hbm, rsd_v)):
        for j in range(NW):
            pltpu.async_copy(h_hbm.at[pl.ds(j * NPAD + base, RPT)],
                             h2_v.at[j], bsem)
        for j in range(NW):
            pltpu.make_async_copy(h_hbm.at[pl.ds(j * NPAD + base, RPT)],
                                  h2_v.at[j], bsem).wait()

        @pl.loop(0, RPT // 16)
        def _(k):
            sl = pl.ds(k * 16, 16)
            a = h2_v[0, sl]
            for j in range(1, NW):
                a = a + h2_v[j, sl]
            rs_v[sl] = _rsqrt16(jnp.maximum(a, 1.0))

    @pl.loop(0, RPT // 16)
    def _(k):
        sl = pl.ds(k * 16, 16)
        cc_v[sl] = rss_v[sl] * rsd_v[sl]

    pltpu.sync_copy(rsd_v, rsd_out.at[pl.ds(base, RPT)])
    pltpu.sync_copy(cc_v, cc_out.at[pl.ds(base, RPT)])

    pltpu.sync_copy(x_hbm.at[pl.ds(base, RPT)], xbuf)

    @pl.loop(0, RPT)
    def _(r):
        sc16 = _splat16(rss_v, r)
        for k2 in range(D // 16):
            sl = pl.ds(k2 * 16, 16)
            xbuf[r, sl] = xbuf[r, sl] * sc16

    pltpu.sync_copy(xbuf, g0_out.at[pl.ds(base, RPT)])


def _prep_call(hs, hd, x_pad):
    f = functools.partial(
        pl.kernel,
        out_type=[jax.ShapeDtypeStruct((NPAD,), _f32),
                  jax.ShapeDtypeStruct((NPAD,), _f32),
                  jax.ShapeDtypeStruct((NPAD, D), _f32)],
        mesh=_mesh(),
        compiler_params=_cparams(),
        scratch_types=[pltpu.VMEM((NW, RPT), _f32),
                       pltpu.VMEM((RPT,), _f32),
                       pltpu.VMEM((RPT,), _f32),
                       pltpu.VMEM((RPT,), _f32),
                       pltpu.VMEM((RPT, D), _f32),
                       pltpu.SemaphoreType.DMA],
    )(_prep_body)
    return f(hs, hd, x_pad)


def _spmm_body(first, last, *refs):
    n_in = 5 + (0 if first else 1) + (1 if last else 0)
    if last:
        g_hbm, ks3, kd3, cnt_hbm, sc_hbm, sp_hbm, x_hbm = refs[:n_in]
        out_hbm, = refs[n_in:n_in + 1]
        scr = refs[n_in + 1:]
    elif first:
        g_hbm, ks3, kd3, cnt_hbm, sc_hbm = refs[:n_in]
        g_out, s_out = refs[n_in:n_in + 2]
        scr = refs[n_in + 2:]
    else:
        g_hbm, ks3, kd3, cnt_hbm, sc_hbm, sp_hbm = refs[:n_in]
        g_out, s_out = refs[n_in:n_in + 2]
        scr = refs[n_in + 2:]
    acc, idx_sv, idx_dv, rowbuf, cc_v, cnt_v, gsem, ssem = scr

    c = lax.axis_index("c")
    s = lax.axis_index("s")
    w = c * NS + s
    zeros = jnp.zeros((16,), _f32)

    @pl.loop(0, KP)
    def _(r):
        for k2 in range(D // 16):
            rowbuf[2, r, pl.ds(k2 * 16, 16)] = zeros

    pltpu.sync_copy(rowbuf.at[2], acc.at[pl.ds(s * RZ, KP)])
    pltpu.sync_copy(rowbuf.at[2], acc.at[pl.ds(s * RZ + KP, KP)])
    pltpu.sync_copy(rowbuf.at[2, pl.ds(0, RZ - 2 * KP)],
                    acc.at[pl.ds(s * RZ + 2 * KP, RZ - 2 * KP)])

    pltpu.sync_copy(ks3.at[w], idx_sv)
    pltpu.sync_copy(kd3.at[w], idx_dv)
    pltpu.sync_copy(cnt_hbm.at[pl.ds(w * 16, 16)], cnt_v)
    nch = cnt_v[pl.ds(0, 16)][0]
    plsc.subcore_barrier()

    def sg(i, b):
        pltpu.async_copy(g_hbm.at[idx_sv.at[i]], rowbuf.at[b], gsem)

    def wg(i, b):
        pltpu.make_async_copy(g_hbm.at[idx_sv.at[i]], rowbuf.at[b],
                              gsem).wait()

    def sa(i, b):
        pltpu.async_copy(rowbuf.at[b], acc.at[idx_dv.at[i]], ssem, add=True)

    def wa(i, b):
        pltpu.make_async_copy(rowbuf.at[b], acc.at[idx_dv.at[i]],
                              ssem).wait()

    @pl.when(nch >= 1)
    def _():
        sg(0, 0)

    @pl.when(nch >= 2)
    def _():
        sg(1, 1)

    @pl.loop(0, nch)
    def _(i):
        b = i % 3
        wg(i, b)
        sa(i, b)

        @pl.when(i >= 1)
        def _():
            wa(i - 1, (i - 1) % 3)

        @pl.when(i + 2 < nch)
        def _():
            sg(i + 2, (i + 2) % 3)

    @pl.when(nch >= 1)
    def _():
        wa(nch - 1, (nch - 1) % 3)

    plsc.subcore_barrier()

    gb = c * NHALF + s * RE
    lb = s * RE
    pltpu.sync_copy(sc_hbm.at[pl.ds(gb, RE)], cc_v)

    off = 0
    for n in (KP, KP, RE - 2 * KP):
        b0 = rowbuf.at[0, pl.ds(0, n)]
        b1 = rowbuf.at[1, pl.ds(0, n)]
        b2 = rowbuf.at[2, pl.ds(0, n)]
        pltpu.sync_copy(acc.at[pl.ds(lb + off, n)], b0)
        if not first:
            pltpu.sync_copy(sp_hbm.at[pl.ds(gb + off, n)], b1)
        if last:
            pltpu.sync_copy(x_hbm.at[pl.ds(gb + off, n)], b2)
        if first and not last:
            pltpu.sync_copy(b0, s_out.at[pl.ds(gb + off, n)])

        off_ = off

        @pl.loop(0, n)
        def _(r):
            sc16 = _splat16(cc_v, off_ + r)
            for k2 in range(D // 16):
                sl = pl.ds(k2 * 16, 16)
                a = rowbuf[0, r, sl]
                if last:
                    t = rowbuf[1, r, sl] + a
                    rowbuf[0, r, sl] = (0.25 * rowbuf[2, r, sl]
                                        + (0.25 * sc16) * t)
                elif first:
                    rowbuf[0, r, sl] = sc16 * a
                else:
                    rowbuf[1, r, sl] = rowbuf[1, r, sl] + a
                    rowbuf[0, r, sl] = sc16 * a

        if last:
            pltpu.sync_copy(b0, out_hbm.at[pl.ds(gb + off, n)])
        else:
            if not first:
                pltpu.sync_copy(b1, s_out.at[pl.ds(gb + off, n)])
            pltpu.sync_copy(b0, g_out.at[pl.ds(gb + off, n)])
        off += n


def _spmm_call(first, last, g, ks3, kd3, cnt, scale, s_prev=None, x_pad=None):
    if last:
        outs = [jax.ShapeDtypeStruct((NPAD, D), _f32)]
    else:
        outs = [jax.ShapeDtypeStruct((NPAD, D), _f32),
                jax.ShapeDtypeStruct((NPAD, D), _f32)]
    f = functools.partial(
        pl.kernel,
        out_type=outs,
        mesh=_mesh(),
        compiler_params=_cparams(),
        scratch_types=[pltpu.VMEM_SHARED((ACCR, D), _f32),
                       pltpu.VMEM((MAXCH, KP), _i32),
                       pltpu.VMEM((MAXCH, KP), _i32),
                       pltpu.VMEM((3, KP, D), _f32),
                       pltpu.VMEM((RE,), _f32),
                       pltpu.VMEM((16,), _i32),
                       pltpu.SemaphoreType.DMA,
                       pltpu.SemaphoreType.DMA],
    )(functools.partial(_spmm_body, first, last))
    args = [g, ks3, kd3, cnt, scale]
    if not first:
        args.append(s_prev)
    if last:
        args.append(x_pad)
    return f(*args)


def kernel(x, edge_index):
    src = edge_index[0].astype(_i32)
    dst = edge_index[1].astype(_i32)
    x_pad = jnp.pad(x, ((0, NPAD - N), (0, 0)))
    srcE = src.reshape(NS, EPT)
    dstE = dst.reshape(NS, EPT)

    hs, hd, ks, kd, cnt = _edges_call(srcE, dstE)
    ks3 = ks.reshape(NW, MAXCH, KP)
    kd3 = kd.reshape(NW, MAXCH, KP)
    rs_dst, c_comb, g0 = _prep_call(hs, hd, x_pad)

    g1, s1 = _spmm_call(True, False, g0, ks3, kd3, cnt, c_comb)
    g2, s2 = _spmm_call(False, False, g1, ks3, kd3, cnt, c_comb, s_prev=s1)
    out_pad = _spmm_call(False, True, g2, ks3, kd3, cnt, rs_dst, s_prev=s2,
                         x_pad=x_pad)[0]
    return out_pad[:N]


# speedup vs baseline: 1.3401x; 1.3401x over previous
import functools

import jax
import jax.numpy as jnp
from jax import lax
from jax.experimental import pallas as pl
from jax.experimental.pallas import tpu as pltpu
from jax.experimental.pallas import tpu_sc as plsc

N = 10000
NPAD = 10240
D = 128
E = 320000
NC = 2
NS = 16
NW = NC * NS
EPW = E // NW
EPT = E // NS
NHALF = NPAD // NC
KP = 120
MAXCH = 168
KEPT = MAXCH * KP
ACCR = NHALF + NS
TRASH = NHALF
RZ = ACCR // NS
RE = NHALF // NS
RPT = NPAD // NW

_f32 = jnp.float32
_i32 = jnp.int32


def _mesh():
    return plsc.VectorSubcoreMesh(core_axis_name="c", subcore_axis_name="s")


def _cparams():
    return pltpu.CompilerParams(needs_layout_passes=False,
                                use_tc_tiling_on_sc=False)


def _wid():
    return lax.axis_index("c") * NS + lax.axis_index("s")


def _rsqrt16(x):
    i = plsc.bitcast(x, _i32)
    i = jnp.int32(0x5F3759DF) - (i >> 1)
    y = plsc.bitcast(i, _f32)
    for _ in range(3):
        y = y * (1.5 - 0.5 * x * y * y)
    return y


def _splat16(vec_ref, idx):
    return plsc.load_gather(vec_ref, [jnp.zeros((16,), _i32) + idx])


def _edges_body(src_hbm, dst_hbm, hs_out, hd_out, ks_out, kd_out, cnt_out,
                src_in, dst_in, hist_v, ks_v, kd_v, cnt_v):
    c = lax.axis_index("c")
    s = lax.axis_index("s")
    w = c * NS + s
    iota = lax.iota(_i32, 16)
    ones = jnp.ones((16,), _f32)
    zeros = jnp.zeros((16,), _f32)
    full = iota >= 0

    pltpu.sync_copy(src_hbm.at[s], src_in)
    pltpu.sync_copy(dst_hbm.at[s], dst_in)

    hoff = c * EPW
    for buf, out_ref in ((src_in, hs_out), (dst_in, hd_out)):
        @pl.loop(0, NPAD // 16)
        def _(j):
            hist_v[pl.ds(j * 16, 16)] = zeros

        @pl.loop(0, EPW // 16)
        def _(g):
            idx16 = buf[pl.ds(hoff + g * 16, 16)]
            for lane in range(16):
                plsc.addupdate_scatter(hist_v, [idx16], ones,
                                       mask=iota == lane)

        pltpu.sync_copy(hist_v, out_ref.at[pl.ds(w * NPAD, NPAD)])

    lo = c * NHALF

    @pl.loop(0, EPT // 16, init_carry=jnp.int32(0))
    def pos(g, pos):
        sl = pl.ds(g * 16, 16)
        s16 = src_in[sl]
        d16 = dst_in[sl]
        mi = ((d16 >= lo) & (d16 < lo + NHALF)).astype(_i32)
        csum = plsc.cumsum(mi)
        dest = pos + csum - mi
        m = mi > 0
        plsc.store_scatter(ks_v, [dest], s16, mask=m)
        plsc.store_scatter(kd_v, [dest], d16 - lo, mask=m)
        return pos + csum[15]

    zi16 = jnp.zeros((16,), _i32)
    trash16 = zi16 + TRASH
    for g in range(9):
        dest = pos + g * 16 + iota
        plsc.store_scatter(ks_v, [dest], zi16, mask=full)
        plsc.store_scatter(kd_v, [dest], trash16, mask=full)

    nch = (pos + (KP - 1)) // KP
    cnt_v[pl.ds(0, 16)] = zi16 + nch
    pltpu.sync_copy(cnt_v, cnt_out.at[pl.ds(w * 16, 16)])
    pltpu.sync_copy(ks_v, ks_out.at[pl.ds(w * KEPT, KEPT)])
    pltpu.sync_copy(kd_v, kd_out.at[pl.ds(w * KEPT, KEPT)])


def _edges_call(srcE, dstE):
    f = functools.partial(
        pl.kernel,
        out_type=[jax.ShapeDtypeStruct((NW * NPAD,), _f32),
                  jax.ShapeDtypeStruct((NW * NPAD,), _f32),
                  jax.ShapeDtypeStruct((NW * KEPT,), _i32),
                  jax.ShapeDtypeStruct((NW * KEPT,), _i32),
                  jax.ShapeDtypeStruct((NW * 16,), _i32)],
        mesh=_mesh(),
        compiler_params=_cparams(),
        scratch_types=[pltpu.VMEM((EPT,), _i32),
                       pltpu.VMEM((EPT,), _i32),
                       pltpu.VMEM((NPAD,), _f32),
                       pltpu.VMEM((KEPT,), _i32),
                       pltpu.VMEM((KEPT,), _i32),
                       pltpu.VMEM((16,), _i32)],
    )(_edges_body)
    return f(srcE, dstE)


def _prep_body(hs_hbm, hd_hbm, x_hbm, rsd_out, cc_out, g0_out,
               h2_v, rss_v, rsd_v, cc_v, xbuf, bsem):
    w = _wid()
    base = w * RPT

    for h_hbm, rs_v in ((hs_hbm, rss_v), (hd_hbm, rsd_v)):
        for j in range(NW):
            pltpu.async_copy(h_hbm.at[pl.ds(j * NPAD + base, RPT)],
                             h2_v.at[j], bsem)
        for j in range(NW):
            pltpu.make_async_copy(h_hbm.at[pl.ds(j * NPAD + base, RPT)],
                                  h2_v.at[j], bsem).wait()

        @pl.loop(0, RPT // 16)
        def _(k):
            sl = pl.ds(k * 16, 16)
            a = h2_v[0, sl]
            for j in range(1, NW):
                a = a + h2_v[j, sl]
            rs_v[sl] = _rsqrt16(jnp.maximum(a, 1.0))

    @pl.loop(0, RPT // 16)
    def _(k):
        sl = pl.ds(k * 16, 16)
        cc_v[sl] = rss_v[sl] * rsd_v[sl]

    pltpu.sync_copy(rsd_v, rsd_out.at[pl.ds(base, RPT)])
    pltpu.sync_copy(cc_v, cc_out.at[pl.ds(base, RPT)])

    pltpu.sync_copy(x_hbm.at[pl.ds(base, RPT)], xbuf)

    @pl.loop(0, RPT)
    def _(r):
        sc16 = _splat16(rss_v, r)
        for k2 in range(D // 16):
            sl = pl.ds(k2 * 16, 16)
            xbuf[r, sl] = xbuf[r, sl] * sc16

    pltpu.sync_copy(xbuf, g0_out.at[pl.ds(base, RPT)])


def _prep_call(hs, hd, x_pad):
    f = functools.partial(
        pl.kernel,
        out_type=[jax.ShapeDtypeStruct((NPAD,), _f32),
                  jax.ShapeDtypeStruct((NPAD,), _f32),
                  jax.ShapeDtypeStruct((NPAD, D), _f32)],
        mesh=_mesh(),
        compiler_params=_cparams(),
        scratch_types=[pltpu.VMEM((NW, RPT), _f32),
                       pltpu.VMEM((RPT,), _f32),
                       pltpu.VMEM((RPT,), _f32),
                       pltpu.VMEM((RPT,), _f32),
                       pltpu.VMEM((RPT, D), _f32),
                       pltpu.SemaphoreType.DMA],
    )(_prep_body)
    return f(hs, hd, x_pad)


def _spmm_body(first, last, *refs):
    n_in = 5 + (0 if first else 1) + (1 if last else 0)
    if last:
        g_hbm, ks3, kd3, cnt_hbm, sc_hbm, sp_hbm, x_hbm = refs[:n_in]
        out_hbm, = refs[n_in:n_in + 1]
        scr = refs[n_in + 1:]
    elif first:
        g_hbm, ks3, kd3, cnt_hbm, sc_hbm = refs[:n_in]
        g_out, s_out = refs[n_in:n_in + 2]
        scr = refs[n_in + 2:]
    else:
        g_hbm, ks3, kd3, cnt_hbm, sc_hbm, sp_hbm = refs[:n_in]
        g_out, s_out = refs[n_in:n_in + 2]
        scr = refs[n_in + 2:]
    acc, idx_sv, idx_dv, rowbuf, cc_v, cnt_v, gsem, ssem = scr

    c = lax.axis_index("c")
    s = lax.axis_index("s")
    w = c * NS + s
    zeros = jnp.zeros((16,), _f32)

    @pl.loop(0, KP)
    def _(r):
        for k2 in range(D // 16):
            rowbuf[2, r, pl.ds(k2 * 16, 16)] = zeros

    pltpu.sync_copy(rowbuf.at[2], acc.at[pl.ds(s * RZ, KP)])
    pltpu.sync_copy(rowbuf.at[2], acc.at[pl.ds(s * RZ + KP, KP)])
    pltpu.sync_copy(rowbuf.at[2, pl.ds(0, RZ - 2 * KP)],
                    acc.at[pl.ds(s * RZ + 2 * KP, RZ - 2 * KP)])

    pltpu.sync_copy(ks3.at[w], idx_sv)
    pltpu.sync_copy(kd3.at[w], idx_dv)
    pltpu.sync_copy(cnt_hbm.at[pl.ds(w * 16, 16)], cnt_v)
    nch = cnt_v[pl.ds(0, 16)][0]
    plsc.subcore_barrier()

    def sg(i, b):
        pltpu.async_copy(g_hbm.at[idx_sv.at[i]], rowbuf.at[b], gsem)

    def wg(i, b):
        pltpu.make_async_copy(g_hbm.at[idx_sv.at[i]], rowbuf.at[b],
                              gsem).wait()

    def sa(i, b):
        pltpu.async_copy(rowbuf.at[b], acc.at[idx_dv.at[i]], ssem, add=True)

    def wa(i, b):
        pltpu.make_async_copy(rowbuf.at[b], acc.at[idx_dv.at[i]],
                              ssem).wait()

    @pl.when(nch >= 1)
    def _():
        sg(0, 0)

    @pl.when(nch >= 2)
    def _():
        sg(1, 1)

    @pl.loop(0, nch)
    def _(i):
        b = i % 3
        wg(i, b)
        sa(i, b)

        @pl.when(i >= 1)
        def _():
            wa(i - 1, (i - 1) % 3)

        @pl.when(i + 2 < nch)
        def _():
            sg(i + 2, (i + 2) % 3)

    @pl.when(nch >= 1)
    def _():
        wa(nch - 1, (nch - 1) % 3)

    plsc.subcore_barrier()

    gb = c * NHALF + s * RE
    lb = s * RE
    pltpu.sync_copy(sc_hbm.at[pl.ds(gb, RE)], cc_v)

    off = 0
    for n in (KP, KP, RE - 2 * KP):
        b0 = rowbuf.at[0, pl.ds(0, n)]
        b1 = rowbuf.at[1, pl.ds(0, n)]
        b2 = rowbuf.at[2, pl.ds(0, n)]
        pltpu.sync_copy(acc.at[pl.ds(lb + off, n)], b0)
        if not first:
            pltpu.sync_copy(sp_hbm.at[pl.ds(gb + off, n)], b1)
        if last:
            pltpu.sync_copy(x_hbm.at[pl.ds(gb + off, n)], b2)
        if first and not last:
            pltpu.sync_copy(b0, s_out.at[pl.ds(gb + off, n)])

        off_ = off

        @pl.loop(0, n)
        def _(r):
            sc16 = _splat16(cc_v, off_ + r)
            for k2 in range(D // 16):
                sl = pl.ds(k2 * 16, 16)
                a = rowbuf[0, r, sl]
                if last:
                    t = rowbuf[1, r, sl] + a
                    rowbuf[0, r, sl] = (0.25 * rowbuf[2, r, sl]
                                        + (0.25 * sc16) * t)
                elif first:
                    rowbuf[0, r, sl] = sc16 * a
                else:
                    rowbuf[1, r, sl] = rowbuf[1, r, sl] + a
                    rowbuf[0, r, sl] = sc16 * a

        if last:
            pltpu.sync_copy(b0, out_hbm.at[pl.ds(gb + off, n)])
        else:
            if not first:
                pltpu.sync_copy(b1, s_out.at[pl.ds(gb + off, n)])
            pltpu.sync_copy(b0, g_out.at[pl.ds(gb + off, n)])
        off += n


def _spmm_call(first, last, g, ks3, kd3, cnt, scale, s_prev=None, x_pad=None):
    if last:
        outs = [jax.ShapeDtypeStruct((NPAD, D), _f32)]
    else:
        outs = [jax.ShapeDtypeStruct((NPAD, D), _f32),
                jax.ShapeDtypeStruct((NPAD, D), _f32)]
    f = functools.partial(
        pl.kernel,
        out_type=outs,
        mesh=_mesh(),
        compiler_params=_cparams(),
        scratch_types=[pltpu.VMEM_SHARED((ACCR, D), _f32),
                       pltpu.VMEM((MAXCH, KP), _i32),
                       pltpu.VMEM((MAXCH, KP), _i32),
                       pltpu.VMEM((3, KP, D), _f32),
                       pltpu.VMEM((RE,), _f32),
                       pltpu.VMEM((16,), _i32),
                       pltpu.SemaphoreType.DMA,
                       pltpu.SemaphoreType.DMA],
    )(functools.partial(_spmm_body, first, last))
    args = [g, ks3, kd3, cnt, scale]
    if not first:
        args.append(s_prev)
    if last:
        args.append(x_pad)
    return f(*args)


def kernel(x, edge_index):
    src = edge_index[0].astype(_i32)
    dst = edge_index[1].astype(_i32)
    x_pad = jnp.pad(x, ((0, NPAD - N), (0, 0)))
    srcE = src.reshape(NS, EPT)
    dstE = dst.reshape(NS, EPT)

    hs, hd, ks, kd, cnt = _edges_call(srcE, dstE)
    ks3 = ks.reshape(NW, MAXCH, KP)
    kd3 = kd.reshape(NW, MAXCH, KP)
    rs_dst, c_comb, g0 = _prep_call(hs, hd, x_pad)

    g1, s1 = _spmm_call(True, False, g0, ks3, kd3, cnt, c_comb)
    g2, s2 = _spmm_call(False, False, g1, ks3, kd3, cnt, c_comb, s_prev=s1)
    out_pad = _spmm_call(False, True, g2, ks3, kd3, cnt, rs_dst, s_prev=s2,
                         x_pad=x_pad)[0]
    return out_pad[:N]
